# baseline (speedup 1.0000x reference)
import jax
import jax.numpy as jnp
from jax.experimental import pallas as pl
from jax.experimental.pallas import tpu as pltpu

M = 1024
K = 150528
N_EXP = 64
K_BLK = 3072
NUM_K = K // K_BLK


def _router_kernel(x_ref, w_ref, b_ref, out_ref, acc_ref):
    k = pl.program_id(0)
    part = jax.lax.dot_general(
        x_ref[...], w_ref[...],
        (((1,), (1,)), ((), ())),
        preferred_element_type=jnp.float32,
    )

    @pl.when(k == 0)
    def _init():
        acc_ref[...] = part + b_ref[...]

    @pl.when(k > 0)
    def _accum():
        acc_ref[...] += part

    @pl.when(k == NUM_K - 1)
    def _finish():
        acc = acc_ref[...]
        iota = jax.lax.broadcasted_iota(jnp.int32, acc.shape, 1)
        mx = jnp.max(acc, axis=1, keepdims=True)
        idx = jnp.min(jnp.where(acc == mx, iota, N_EXP),
                      axis=1, keepdims=True)
        out_ref[...] = idx


def kernel(x, W, b):
    x_flat = x.reshape(M, K)
    b2 = b.reshape(1, N_EXP)
    out = pl.pallas_call(
        _router_kernel,
        grid=(NUM_K,),
        in_specs=[
            pl.BlockSpec((M, K_BLK), lambda k: (0, k)),
            pl.BlockSpec((N_EXP, K_BLK), lambda k: (0, k)),
            pl.BlockSpec((1, N_EXP), lambda k: (0, 0)),
        ],
        out_specs=pl.BlockSpec((M, 1), lambda k: (0, 0)),
        out_shape=jax.ShapeDtypeStruct((M, 1), jnp.int32),
        scratch_shapes=[pltpu.VMEM((M, N_EXP), jnp.float32)],
        compiler_params=pltpu.CompilerParams(
            dimension_semantics=("arbitrary",),
        ),
    )(x_flat, W, b2)
    return out.reshape(M)
